# baseline (speedup 1.0000x reference)
import functools
import math

import jax
import jax.numpy as jnp
from jax.experimental import pallas as pl
from jax.experimental.pallas import tpu as pltpu
from jax.experimental.pallas import tpu_sc as plsc

_COMMIT = 0.25
_STRIDE1, _STRIDE2 = 4, 16


def _build_point_features(coords, timestamps, mask):
    valid = mask[..., None]
    lat = coords[..., 0:1]
    lon = coords[..., 1:2]
    denom = jnp.clip(jnp.sum(valid, axis=1, keepdims=True), 1.0, None)
    mean_lat = jnp.sum(lat * valid, axis=1, keepdims=True) / denom
    mean_lon = jnp.sum(lon * valid, axis=1, keepdims=True) / denom
    lat_rel = (lat - mean_lat) * valid
    lon_rel = (lon - mean_lon) * valid
    z1 = jnp.zeros_like(lat[:, :1])
    dlat = jnp.concatenate([z1, lat[:, 1:] - lat[:, :-1]], axis=1) * valid
    dlon = jnp.concatenate([z1, lon[:, 1:] - lon[:, :-1]], axis=1) * valid
    speed = jnp.sqrt(dlat ** 2 + dlon ** 2)
    heading_lat = dlat / (speed + 1e-06)
    heading_lon = dlon / (speed + 1e-06)
    ts = timestamps.astype(jnp.float32)
    zt = jnp.zeros_like(ts[:, :1])
    dt = jnp.concatenate([zt, jnp.clip(ts[:, 1:] - ts[:, :-1], 0.0, None)], axis=1) * mask
    valid_dt = (dt > 0).astype(jnp.float32)
    mean_dt = jnp.clip(jnp.sum(dt, axis=1, keepdims=True) / jnp.clip(jnp.sum(valid_dt, axis=1, keepdims=True), 1.0, None), 0.001, None)
    dt_norm = dt / mean_dt
    log_dt = jnp.log1p(dt)
    day = 24 * 60 * 60
    week = 7 * day
    t_day = (ts % day) / day
    t_week = (ts % week) / week
    cyc = [jnp.sin(2 * math.pi * t_day)[..., None], jnp.cos(2 * math.pi * t_day)[..., None],
           jnp.sin(2 * math.pi * t_week)[..., None], jnp.cos(2 * math.pi * t_week)[..., None]]
    speed_per_dt = speed / (dt[..., None] + 0.001)
    za = jnp.zeros_like(speed_per_dt[:, :1])
    accel = jnp.concatenate([za, speed_per_dt[:, 1:] - speed_per_dt[:, :-1]], axis=1) * valid
    feats = [lat * valid, lon * valid, lat_rel, lon_rel, dlat, dlon, speed * valid,
             heading_lat * valid, heading_lon * valid, log_dt[..., None], dt_norm[..., None],
             speed_per_dt * valid, accel] + cyc
    return jnp.concatenate(feats, axis=-1)


def _pool(feats, stride):
    b, s, d = feats.shape
    pad = (stride - s % stride) % stride
    if pad:
        feats = jnp.concatenate([feats, jnp.repeat(feats[:, -1:], pad, axis=1)], axis=1)
    return feats.reshape(b, -1, stride, d).mean(axis=2)


def _upsample(tokens, target_len):
    b, s = tokens.shape
    if s == target_len:
        return tokens
    rep = -(-target_len // s)
    return jnp.repeat(tokens, rep, axis=1)[:, :target_len]


def _vq_body(nchunks, chunk, mode, z_ref, zn_ref, cb_ref, cbn_ref, idx_ref, md_ref):
    z = z_ref[...]
    rb = z.shape[0]
    best_d = jnp.full((rb,), jnp.inf, dtype=jnp.float32)
    best_i = jnp.zeros((rb,), dtype=jnp.int32)
    for k in range(nchunks):
        c = cb_ref[k * chunk:(k + 1) * chunk, :]
        if mode == "bf16":
            scores = jax.lax.dot_general(
                z.astype(jnp.bfloat16), c.astype(jnp.bfloat16),
                (((1,), (1,)), ((), ())),
                preferred_element_type=jnp.float32)
        elif mode == "swap":
            scores = jax.lax.dot_general(
                c, z, (((1,), (1,)), ((), ())),
                preferred_element_type=jnp.float32).T
        elif mode == "lhs_nn":
            scores = jax.lax.dot_general(
                z, c.T, (((1,), (0,)), ((), ())),
                preferred_element_type=jnp.float32)
        else:
            scores = jax.lax.dot_general(
                z, c, (((1,), (1,)), ((), ())),
                preferred_element_type=jnp.float32)
        d = (zn_ref[...] - 2.0 * scores) + cbn_ref[0:1, k * chunk:(k + 1) * chunk]
        dmin = jnp.min(d, axis=1)
        cols = jax.lax.broadcasted_iota(jnp.int32, (rb, chunk), 1)
        amin = jnp.min(jnp.where(d == dmin[:, None], cols, chunk), axis=1) + k * chunk
        upd = dmin < best_d
        best_i = jnp.where(upd, amin, best_i)
        best_d = jnp.where(upd, dmin, best_d)
    idx_ref[...] = best_i
    md_ref[...] = best_d


def _vq_argmin(flat, znorm, cb, cbnorm, row_block, chunk, mode="rhs_t"):
    r, d = flat.shape
    n = cb.shape[0]
    nchunks = n // chunk
    body = functools.partial(_vq_body, nchunks, chunk, mode)
    idx, md = pl.pallas_call(
        body,
        grid=(r // row_block,),
        in_specs=[
            pl.BlockSpec((row_block, d), lambda i: (i, 0)),
            pl.BlockSpec((row_block, 1), lambda i: (i, 0)),
            pl.BlockSpec((n, d), lambda i: (0, 0)),
            pl.BlockSpec((1, n), lambda i: (0, 0)),
        ],
        out_specs=[
            pl.BlockSpec((row_block,), lambda i: (i,)),
            pl.BlockSpec((row_block,), lambda i: (i,)),
        ],
        out_shape=[
            jax.ShapeDtypeStruct((r,), jnp.int32),
            jax.ShapeDtypeStruct((r,), jnp.float32),
        ],
        compiler_params=pltpu.CompilerParams(
            dimension_semantics=("parallel",)),
    )(flat, znorm, cb, cbnorm.reshape(1, n))
    return idx, md


_GATHER_WINDOW = 128


def _sc_gather(cb, idx_flat):
    r = idx_flat.shape[0]
    d = cb.shape[1]
    idx2 = idx_flat.reshape(1, r)
    mesh = plsc.VectorSubcoreMesh(core_axis_name="core", subcore_axis_name="subcore")

    @pl.kernel(out_type=jax.ShapeDtypeStruct((r, d), cb.dtype), mesh=mesh)
    def gather_kernel(cb_hbm, i_hbm, o_hbm):
        def body(i_vmem, o_vmem):
            pltpu.sync_copy(cb_hbm.at[i_vmem.at[0]], o_vmem)

        pltpu.emit_pipeline(
            body,
            grid=(r // _GATHER_WINDOW,),
            in_specs=[pl.BlockSpec((1, _GATHER_WINDOW), index_map=lambda i: (0, i))],
            out_specs=[pl.BlockSpec((_GATHER_WINDOW, d), index_map=lambda i: (i, 0))],
            core_axis_name=("core", "subcore"),
            dimension_semantics=(pltpu.PARALLEL,),
        )(i_hbm, o_hbm)

    return gather_kernel(cb, idx2)


def _level_loss(md, numel):
    m = jnp.sum(md) / numel
    return _COMMIT * m + m


def kernel(coords, timestamps, attention_mask, W1, b1, W2, b2, cb0, cb1, cb2):
    feats = _build_point_features(coords, timestamps, attention_mask)
    h = jax.nn.gelu(feats @ W1 + b1, approximate=False)
    z = h @ W2 + b2
    b, t, d = z.shape

    z1 = _pool(z, _STRIDE1)
    z2 = _pool(z, _STRIDE2)
    f0 = z.reshape(-1, d)
    f1 = z1.reshape(-1, d)
    f2 = z2.reshape(-1, d)
    n0 = jnp.sum(f0 ** 2, axis=1, keepdims=True)
    n1 = jnp.sum(f1 ** 2, axis=1, keepdims=True)
    n2 = jnp.sum(f2 ** 2, axis=1, keepdims=True)
    c0n = jnp.sum(cb0 ** 2, axis=1)
    c1n = jnp.sum(cb1 ** 2, axis=1)
    c2n = jnp.sum(cb2 ** 2, axis=1)

    idx0f, md0 = _vq_argmin(f0, n0, cb0, c0n, row_block=512, chunk=2048)
    idx2f, md2 = _vq_argmin(f2, n2, cb2, c2n, row_block=512, chunk=512)

    dist1 = n1 - 2.0 * jax.lax.dot_general(
        f1, cb1, (((1,), (1,)), ((), ())),
        preferred_element_type=jnp.float32) + c1n
    idx1f = jnp.argmin(dist1, axis=1)
    q1 = jnp.take(cb1, idx1f, axis=0)
    l1 = _COMMIT * jnp.mean((q1 - f1) ** 2) + jnp.mean((q1 - f1) ** 2)

    g0 = _sc_gather(cb0, idx0f)
    q0 = (f0 + (g0 - f0)).reshape(z.shape)

    idx0 = idx0f.reshape(b, t)
    tok1 = _upsample(idx1f.reshape(b, -1), t)
    tok2 = _upsample(idx2f.reshape(b, -1), t)

    loss = _level_loss(md0, f0.size) + l1 + _level_loss(md2, f2.size)
    return q0, idx0, tok1, tok2, loss


# speedup vs baseline: 1.0010x; 1.0010x over previous
import functools
import math

import jax
import jax.numpy as jnp
from jax.experimental import pallas as pl
from jax.experimental.pallas import tpu as pltpu
from jax.experimental.pallas import tpu_sc as plsc

_COMMIT = 0.25
_STRIDE1, _STRIDE2 = 4, 16


def _build_point_features(coords, timestamps, mask):
    valid = mask[..., None]
    lat = coords[..., 0:1]
    lon = coords[..., 1:2]
    denom = jnp.clip(jnp.sum(valid, axis=1, keepdims=True), 1.0, None)
    mean_lat = jnp.sum(lat * valid, axis=1, keepdims=True) / denom
    mean_lon = jnp.sum(lon * valid, axis=1, keepdims=True) / denom
    lat_rel = (lat - mean_lat) * valid
    lon_rel = (lon - mean_lon) * valid
    z1 = jnp.zeros_like(lat[:, :1])
    dlat = jnp.concatenate([z1, lat[:, 1:] - lat[:, :-1]], axis=1) * valid
    dlon = jnp.concatenate([z1, lon[:, 1:] - lon[:, :-1]], axis=1) * valid
    speed = jnp.sqrt(dlat ** 2 + dlon ** 2)
    heading_lat = dlat / (speed + 1e-06)
    heading_lon = dlon / (speed + 1e-06)
    ts = timestamps.astype(jnp.float32)
    zt = jnp.zeros_like(ts[:, :1])
    dt = jnp.concatenate([zt, jnp.clip(ts[:, 1:] - ts[:, :-1], 0.0, None)], axis=1) * mask
    valid_dt = (dt > 0).astype(jnp.float32)
    mean_dt = jnp.clip(jnp.sum(dt, axis=1, keepdims=True) / jnp.clip(jnp.sum(valid_dt, axis=1, keepdims=True), 1.0, None), 0.001, None)
    dt_norm = dt / mean_dt
    log_dt = jnp.log1p(dt)
    day = 24 * 60 * 60
    week = 7 * day
    t_day = (ts % day) / day
    t_week = (ts % week) / week
    cyc = [jnp.sin(2 * math.pi * t_day)[..., None], jnp.cos(2 * math.pi * t_day)[..., None],
           jnp.sin(2 * math.pi * t_week)[..., None], jnp.cos(2 * math.pi * t_week)[..., None]]
    speed_per_dt = speed / (dt[..., None] + 0.001)
    za = jnp.zeros_like(speed_per_dt[:, :1])
    accel = jnp.concatenate([za, speed_per_dt[:, 1:] - speed_per_dt[:, :-1]], axis=1) * valid
    feats = [lat * valid, lon * valid, lat_rel, lon_rel, dlat, dlon, speed * valid,
             heading_lat * valid, heading_lon * valid, log_dt[..., None], dt_norm[..., None],
             speed_per_dt * valid, accel] + cyc
    return jnp.concatenate(feats, axis=-1)


def _pool(feats, stride):
    b, s, d = feats.shape
    pad = (stride - s % stride) % stride
    if pad:
        feats = jnp.concatenate([feats, jnp.repeat(feats[:, -1:], pad, axis=1)], axis=1)
    return feats.reshape(b, -1, stride, d).mean(axis=2)


def _upsample(tokens, target_len):
    b, s = tokens.shape
    if s == target_len:
        return tokens
    rep = -(-target_len // s)
    return jnp.repeat(tokens, rep, axis=1)[:, :target_len]


def _vq_body(nchunks, chunk, mode, z_ref, zn_ref, cb_ref, cbn_ref, idx_ref, md_ref):
    z = z_ref[...]
    rb = z.shape[0]
    best_d = jnp.full((rb,), jnp.inf, dtype=jnp.float32)
    best_i = jnp.zeros((rb,), dtype=jnp.int32)
    for k in range(nchunks):
        c = cb_ref[k * chunk:(k + 1) * chunk, :]
        if mode == "bf16":
            scores = jax.lax.dot_general(
                z.astype(jnp.bfloat16), c.astype(jnp.bfloat16),
                (((1,), (1,)), ((), ())),
                preferred_element_type=jnp.float32)
        elif mode == "swap":
            scores = jax.lax.dot_general(
                c, z, (((1,), (1,)), ((), ())),
                preferred_element_type=jnp.float32).T
        elif mode == "lhs_nn":
            scores = jax.lax.dot_general(
                z, c.T, (((1,), (0,)), ((), ())),
                preferred_element_type=jnp.float32)
        else:
            scores = jax.lax.dot_general(
                z, c, (((1,), (1,)), ((), ())),
                preferred_element_type=jnp.float32)
        d = (zn_ref[...] - 2.0 * scores) + cbn_ref[0:1, k * chunk:(k + 1) * chunk]
        dmin = jnp.min(d, axis=1)
        cols = jax.lax.broadcasted_iota(jnp.int32, (rb, chunk), 1)
        amin = jnp.min(jnp.where(d == dmin[:, None], cols, chunk), axis=1) + k * chunk
        upd = dmin < best_d
        best_i = jnp.where(upd, amin, best_i)
        best_d = jnp.where(upd, dmin, best_d)
    idx_ref[...] = best_i
    md_ref[...] = best_d


def _vq_argmin(flat, znorm, cb, cbnorm, row_block, chunk, mode="rhs_t"):
    r, d = flat.shape
    n = cb.shape[0]
    nchunks = n // chunk
    body = functools.partial(_vq_body, nchunks, chunk, mode)
    idx, md = pl.pallas_call(
        body,
        grid=(r // row_block,),
        in_specs=[
            pl.BlockSpec((row_block, d), lambda i: (i, 0)),
            pl.BlockSpec((row_block, 1), lambda i: (i, 0)),
            pl.BlockSpec((n, d), lambda i: (0, 0)),
            pl.BlockSpec((1, n), lambda i: (0, 0)),
        ],
        out_specs=[
            pl.BlockSpec((row_block,), lambda i: (i,)),
            pl.BlockSpec((row_block,), lambda i: (i,)),
        ],
        out_shape=[
            jax.ShapeDtypeStruct((r,), jnp.int32),
            jax.ShapeDtypeStruct((r,), jnp.float32),
        ],
        compiler_params=pltpu.CompilerParams(
            dimension_semantics=("parallel",)),
    )(flat, znorm, cb, cbnorm.reshape(1, n))
    return idx, md


_GATHER_WINDOW = 256


def _sc_gather(cb, idx_flat):
    r = idx_flat.shape[0]
    d = cb.shape[1]
    idx2 = idx_flat.reshape(1, r)
    mesh = plsc.VectorSubcoreMesh(core_axis_name="core", subcore_axis_name="subcore")

    @pl.kernel(out_type=jax.ShapeDtypeStruct((r, d), cb.dtype), mesh=mesh)
    def gather_kernel(cb_hbm, i_hbm, o_hbm):
        def body(i_vmem, o_vmem):
            pltpu.sync_copy(cb_hbm.at[i_vmem.at[0]], o_vmem)

        pltpu.emit_pipeline(
            body,
            grid=(r // _GATHER_WINDOW,),
            in_specs=[pl.BlockSpec((1, _GATHER_WINDOW), index_map=lambda i: (0, i))],
            out_specs=[pl.BlockSpec((_GATHER_WINDOW, d), index_map=lambda i: (i, 0))],
            core_axis_name=("core", "subcore"),
            dimension_semantics=(pltpu.PARALLEL,),
        )(i_hbm, o_hbm)

    return gather_kernel(cb, idx2)


def _level_loss(md, numel):
    m = jnp.sum(md) / numel
    return _COMMIT * m + m


def kernel(coords, timestamps, attention_mask, W1, b1, W2, b2, cb0, cb1, cb2):
    feats = _build_point_features(coords, timestamps, attention_mask)
    h = jax.nn.gelu(feats @ W1 + b1, approximate=False)
    z = h @ W2 + b2
    b, t, d = z.shape

    z1 = _pool(z, _STRIDE1)
    z2 = _pool(z, _STRIDE2)
    f0 = z.reshape(-1, d)
    f1 = z1.reshape(-1, d)
    f2 = z2.reshape(-1, d)
    n0 = jnp.sum(f0 ** 2, axis=1, keepdims=True)
    n1 = jnp.sum(f1 ** 2, axis=1, keepdims=True)
    n2 = jnp.sum(f2 ** 2, axis=1, keepdims=True)
    c0n = jnp.sum(cb0 ** 2, axis=1)
    c1n = jnp.sum(cb1 ** 2, axis=1)
    c2n = jnp.sum(cb2 ** 2, axis=1)

    idx0f, md0 = _vq_argmin(f0, n0, cb0, c0n, row_block=512, chunk=2048)
    idx2f, md2 = _vq_argmin(f2, n2, cb2, c2n, row_block=512, chunk=512)

    dist1 = n1 - 2.0 * jax.lax.dot_general(
        f1, cb1, (((1,), (1,)), ((), ())),
        preferred_element_type=jnp.float32) + c1n
    idx1f = jnp.argmin(dist1, axis=1)
    q1 = jnp.take(cb1, idx1f, axis=0)
    l1 = _COMMIT * jnp.mean((q1 - f1) ** 2) + jnp.mean((q1 - f1) ** 2)

    g0 = _sc_gather(cb0, idx0f)
    q0 = (f0 + (g0 - f0)).reshape(z.shape)

    idx0 = idx0f.reshape(b, t)
    tok1 = _upsample(idx1f.reshape(b, -1), t)
    tok2 = _upsample(idx2f.reshape(b, -1), t)

    loss = _level_loss(md0, f0.size) + l1 + _level_loss(md2, f2.size)
    return q0, idx0, tok1, tok2, loss


# speedup vs baseline: 1.0978x; 1.0967x over previous
import functools
import math

import jax
import jax.numpy as jnp
from jax.experimental import pallas as pl
from jax.experimental.pallas import tpu as pltpu
from jax.experimental.pallas import tpu_sc as plsc

_COMMIT = 0.25
_STRIDE1, _STRIDE2 = 4, 16


def _build_point_features(coords, timestamps, mask):
    valid = mask[..., None]
    lat = coords[..., 0:1]
    lon = coords[..., 1:2]
    denom = jnp.clip(jnp.sum(valid, axis=1, keepdims=True), 1.0, None)
    mean_lat = jnp.sum(lat * valid, axis=1, keepdims=True) / denom
    mean_lon = jnp.sum(lon * valid, axis=1, keepdims=True) / denom
    lat_rel = (lat - mean_lat) * valid
    lon_rel = (lon - mean_lon) * valid
    z1 = jnp.zeros_like(lat[:, :1])
    dlat = jnp.concatenate([z1, lat[:, 1:] - lat[:, :-1]], axis=1) * valid
    dlon = jnp.concatenate([z1, lon[:, 1:] - lon[:, :-1]], axis=1) * valid
    speed = jnp.sqrt(dlat ** 2 + dlon ** 2)
    heading_lat = dlat / (speed + 1e-06)
    heading_lon = dlon / (speed + 1e-06)
    ts = timestamps.astype(jnp.float32)
    zt = jnp.zeros_like(ts[:, :1])
    dt = jnp.concatenate([zt, jnp.clip(ts[:, 1:] - ts[:, :-1], 0.0, None)], axis=1) * mask
    valid_dt = (dt > 0).astype(jnp.float32)
    mean_dt = jnp.clip(jnp.sum(dt, axis=1, keepdims=True) / jnp.clip(jnp.sum(valid_dt, axis=1, keepdims=True), 1.0, None), 0.001, None)
    dt_norm = dt / mean_dt
    log_dt = jnp.log1p(dt)
    day = 24 * 60 * 60
    week = 7 * day
    t_day = (ts % day) / day
    t_week = (ts % week) / week
    cyc = [jnp.sin(2 * math.pi * t_day)[..., None], jnp.cos(2 * math.pi * t_day)[..., None],
           jnp.sin(2 * math.pi * t_week)[..., None], jnp.cos(2 * math.pi * t_week)[..., None]]
    speed_per_dt = speed / (dt[..., None] + 0.001)
    za = jnp.zeros_like(speed_per_dt[:, :1])
    accel = jnp.concatenate([za, speed_per_dt[:, 1:] - speed_per_dt[:, :-1]], axis=1) * valid
    feats = [lat * valid, lon * valid, lat_rel, lon_rel, dlat, dlon, speed * valid,
             heading_lat * valid, heading_lon * valid, log_dt[..., None], dt_norm[..., None],
             speed_per_dt * valid, accel] + cyc
    return jnp.concatenate(feats, axis=-1)


def _pool(feats, stride):
    b, s, d = feats.shape
    pad = (stride - s % stride) % stride
    if pad:
        feats = jnp.concatenate([feats, jnp.repeat(feats[:, -1:], pad, axis=1)], axis=1)
    return feats.reshape(b, -1, stride, d).mean(axis=2)


def _upsample(tokens, target_len):
    b, s = tokens.shape
    if s == target_len:
        return tokens
    rep = -(-target_len // s)
    return jnp.repeat(tokens, rep, axis=1)[:, :target_len]


def _vq_body(nchunks, chunk, zneg_ref, zn_ref, cb_ref, cbn_ref, idx_ref, md_ref):
    zneg = zneg_ref[...]
    rb = zneg.shape[0]
    best_d = jnp.full((rb,), jnp.inf, dtype=jnp.float32)
    best_i = jnp.zeros((rb,), dtype=jnp.int32)
    for k in range(nchunks):
        c = cb_ref[k * chunk:(k + 1) * chunk, :]
        scores = jax.lax.dot_general(
            zneg, c, (((1,), (1,)), ((), ())),
            preferred_element_type=jnp.float32)
        d = (zn_ref[...] + scores) + cbn_ref[0:1, k * chunk:(k + 1) * chunk]
        dmin = jnp.min(d, axis=1)
        cols = jax.lax.broadcasted_iota(jnp.int32, (rb, chunk), 1)
        amin = jnp.min(jnp.where(d == dmin[:, None], cols, chunk), axis=1) + k * chunk
        upd = dmin < best_d
        best_i = jnp.where(upd, amin, best_i)
        best_d = jnp.where(upd, dmin, best_d)
    idx_ref[...] = best_i
    md_ref[...] = best_d


def _vq_argmin(zneg, znorm, cb, cbnorm, row_block, chunk):
    r, d = zneg.shape
    n = cb.shape[0]
    nchunks = n // chunk
    body = functools.partial(_vq_body, nchunks, chunk)
    idx, md = pl.pallas_call(
        body,
        grid=(r // row_block,),
        in_specs=[
            pl.BlockSpec((row_block, d), lambda i: (i, 0)),
            pl.BlockSpec((row_block, 1), lambda i: (i, 0)),
            pl.BlockSpec((n, d), lambda i: (0, 0)),
            pl.BlockSpec((1, n), lambda i: (0, 0)),
        ],
        out_specs=[
            pl.BlockSpec((row_block,), lambda i: (i,)),
            pl.BlockSpec((row_block,), lambda i: (i,)),
        ],
        out_shape=[
            jax.ShapeDtypeStruct((r,), jnp.int32),
            jax.ShapeDtypeStruct((r,), jnp.float32),
        ],
        compiler_params=pltpu.CompilerParams(
            dimension_semantics=("parallel",)),
    )(zneg, znorm, cb, cbnorm.reshape(1, n))
    return idx, md


_GATHER_WINDOW = 256


def _sc_gather(cb, idx_flat):
    r = idx_flat.shape[0]
    d = cb.shape[1]
    idx2 = idx_flat.reshape(1, r)
    mesh = plsc.VectorSubcoreMesh(core_axis_name="core", subcore_axis_name="subcore")

    @pl.kernel(out_type=jax.ShapeDtypeStruct((r, d), cb.dtype), mesh=mesh)
    def gather_kernel(cb_hbm, i_hbm, o_hbm):
        def body(i_vmem, o_vmem):
            pltpu.sync_copy(cb_hbm.at[i_vmem.at[0]], o_vmem)

        pltpu.emit_pipeline(
            body,
            grid=(r // _GATHER_WINDOW,),
            in_specs=[pl.BlockSpec((1, _GATHER_WINDOW), index_map=lambda i: (0, i))],
            out_specs=[pl.BlockSpec((_GATHER_WINDOW, d), index_map=lambda i: (i, 0))],
            core_axis_name=("core", "subcore"),
            dimension_semantics=(pltpu.PARALLEL,),
        )(i_hbm, o_hbm)

    return gather_kernel(cb, idx2)


def _level_loss(md, numel):
    m = jnp.sum(md) / numel
    return _COMMIT * m + m


def kernel(coords, timestamps, attention_mask, W1, b1, W2, b2, cb0, cb1, cb2):
    feats = _build_point_features(coords, timestamps, attention_mask)
    h = jax.nn.gelu(feats @ W1 + b1, approximate=False)
    z = h @ W2 + b2
    b, t, d = z.shape

    z1 = _pool(z, _STRIDE1)
    z2 = _pool(z, _STRIDE2)
    f0 = z.reshape(-1, d)
    f1 = z1.reshape(-1, d)
    f2 = z2.reshape(-1, d)
    n0 = jnp.sum(f0 ** 2, axis=1, keepdims=True)
    n1 = jnp.sum(f1 ** 2, axis=1, keepdims=True)
    n2 = jnp.sum(f2 ** 2, axis=1, keepdims=True)
    c0n = jnp.sum(cb0 ** 2, axis=1)
    c1n = jnp.sum(cb1 ** 2, axis=1)
    c2n = jnp.sum(cb2 ** 2, axis=1)

    zneg0 = -2.0 * f0
    zneg2 = -2.0 * f2

    half = f0.shape[0] // 2
    idx0a, md0a = _vq_argmin(zneg0[:half], n0[:half], cb0, c0n,
                             row_block=512, chunk=2048)
    g0a = _sc_gather(cb0, idx0a)
    idx0b, md0b = _vq_argmin(zneg0[half:], n0[half:], cb0, c0n,
                             row_block=512, chunk=2048)
    g0b = _sc_gather(cb0, idx0b)
    idx0f = jnp.concatenate([idx0a, idx0b])
    g0 = jnp.concatenate([g0a, g0b])

    idx2f, md2 = _vq_argmin(zneg2, n2, cb2, c2n, row_block=512, chunk=512)

    dist1 = n1 - 2.0 * jax.lax.dot_general(
        f1, cb1, (((1,), (1,)), ((), ())),
        preferred_element_type=jnp.float32) + c1n
    idx1f = jnp.argmin(dist1, axis=1)
    q1 = jnp.take(cb1, idx1f, axis=0)
    l1 = _COMMIT * jnp.mean((q1 - f1) ** 2) + jnp.mean((q1 - f1) ** 2)

    q0 = (f0 + (g0 - f0)).reshape(z.shape)

    idx0 = idx0f.reshape(b, t)
    tok1 = _upsample(idx1f.reshape(b, -1), t)
    tok2 = _upsample(idx2f.reshape(b, -1), t)

    m0 = (jnp.sum(md0a) + jnp.sum(md0b)) / f0.size
    l0 = _COMMIT * m0 + m0
    loss = l0 + l1 + _level_loss(md2, f2.size)
    return q0, idx0, tok1, tok2, loss


# speedup vs baseline: 1.1851x; 1.0796x over previous
import functools
import math

import jax
import jax.numpy as jnp
from jax.experimental import pallas as pl
from jax.experimental.pallas import tpu as pltpu
from jax.experimental.pallas import tpu_sc as plsc

_COMMIT = 0.25
_STRIDE1, _STRIDE2 = 4, 16


def _build_point_features(coords, timestamps, mask):
    valid = mask[..., None]
    lat = coords[..., 0:1]
    lon = coords[..., 1:2]
    denom = jnp.clip(jnp.sum(valid, axis=1, keepdims=True), 1.0, None)
    mean_lat = jnp.sum(lat * valid, axis=1, keepdims=True) / denom
    mean_lon = jnp.sum(lon * valid, axis=1, keepdims=True) / denom
    lat_rel = (lat - mean_lat) * valid
    lon_rel = (lon - mean_lon) * valid
    z1 = jnp.zeros_like(lat[:, :1])
    dlat = jnp.concatenate([z1, lat[:, 1:] - lat[:, :-1]], axis=1) * valid
    dlon = jnp.concatenate([z1, lon[:, 1:] - lon[:, :-1]], axis=1) * valid
    speed = jnp.sqrt(dlat ** 2 + dlon ** 2)
    heading_lat = dlat / (speed + 1e-06)
    heading_lon = dlon / (speed + 1e-06)
    ts = timestamps.astype(jnp.float32)
    zt = jnp.zeros_like(ts[:, :1])
    dt = jnp.concatenate([zt, jnp.clip(ts[:, 1:] - ts[:, :-1], 0.0, None)], axis=1) * mask
    valid_dt = (dt > 0).astype(jnp.float32)
    mean_dt = jnp.clip(jnp.sum(dt, axis=1, keepdims=True) / jnp.clip(jnp.sum(valid_dt, axis=1, keepdims=True), 1.0, None), 0.001, None)
    dt_norm = dt / mean_dt
    log_dt = jnp.log1p(dt)
    day = 24 * 60 * 60
    week = 7 * day
    t_day = (ts % day) / day
    t_week = (ts % week) / week
    cyc = [jnp.sin(2 * math.pi * t_day)[..., None], jnp.cos(2 * math.pi * t_day)[..., None],
           jnp.sin(2 * math.pi * t_week)[..., None], jnp.cos(2 * math.pi * t_week)[..., None]]
    speed_per_dt = speed / (dt[..., None] + 0.001)
    za = jnp.zeros_like(speed_per_dt[:, :1])
    accel = jnp.concatenate([za, speed_per_dt[:, 1:] - speed_per_dt[:, :-1]], axis=1) * valid
    feats = [lat * valid, lon * valid, lat_rel, lon_rel, dlat, dlon, speed * valid,
             heading_lat * valid, heading_lon * valid, log_dt[..., None], dt_norm[..., None],
             speed_per_dt * valid, accel] + cyc
    return jnp.concatenate(feats, axis=-1)


def _pool(feats, stride):
    b, s, d = feats.shape
    pad = (stride - s % stride) % stride
    if pad:
        feats = jnp.concatenate([feats, jnp.repeat(feats[:, -1:], pad, axis=1)], axis=1)
    return feats.reshape(b, -1, stride, d).mean(axis=2)


def _upsample(tokens, target_len):
    b, s = tokens.shape
    if s == target_len:
        return tokens
    rep = -(-target_len // s)
    return jnp.repeat(tokens, rep, axis=1)[:, :target_len]


def _vq_body(nchunks, chunk, zneg_ref, zn_ref, cb_ref, cbn_ref, idx_ref, md_ref):
    zneg = zneg_ref[...]
    rb = zneg.shape[0]
    best_d = jnp.full((rb,), jnp.inf, dtype=jnp.float32)
    best_i = jnp.zeros((rb,), dtype=jnp.int32)
    for k in range(nchunks):
        c = cb_ref[k * chunk:(k + 1) * chunk, :]
        scores = jax.lax.dot_general(
            zneg, c, (((1,), (1,)), ((), ())),
            preferred_element_type=jnp.float32)
        d = (zn_ref[...] + scores) + cbn_ref[0:1, k * chunk:(k + 1) * chunk]
        dmin = jnp.min(d, axis=1)
        cols = jax.lax.broadcasted_iota(jnp.int32, (rb, chunk), 1)
        amin = jnp.min(jnp.where(d == dmin[:, None], cols, chunk), axis=1) + k * chunk
        upd = dmin < best_d
        best_i = jnp.where(upd, amin, best_i)
        best_d = jnp.where(upd, dmin, best_d)
    idx_ref[...] = best_i
    md_ref[...] = best_d


def _vq_argmin(zneg, znorm, cb, cbnorm, row_block, chunk,
               row_start=0, row_count=None):
    r, d = zneg.shape
    if row_count is None:
        row_count = r
    off = row_start // row_block
    n = cb.shape[0]
    nchunks = n // chunk
    body = functools.partial(_vq_body, nchunks, chunk)
    idx, md = pl.pallas_call(
        body,
        grid=(row_count // row_block,),
        in_specs=[
            pl.BlockSpec((row_block, d), lambda i: (i + off, 0)),
            pl.BlockSpec((row_block, 1), lambda i: (i + off, 0)),
            pl.BlockSpec((n, d), lambda i: (0, 0)),
            pl.BlockSpec((1, n), lambda i: (0, 0)),
        ],
        out_specs=[
            pl.BlockSpec((row_block,), lambda i: (i,)),
            pl.BlockSpec((row_block,), lambda i: (i,)),
        ],
        out_shape=[
            jax.ShapeDtypeStruct((row_count,), jnp.int32),
            jax.ShapeDtypeStruct((row_count,), jnp.float32),
        ],
        compiler_params=pltpu.CompilerParams(
            dimension_semantics=("parallel",)),
    )(zneg, znorm, cb, cbnorm.reshape(1, n))
    return idx, md


_GATHER_WINDOW = 256


def _sc_gather(cb, idx_flat):
    r = idx_flat.shape[0]
    d = cb.shape[1]
    idx2 = idx_flat.reshape(1, r)
    mesh = plsc.VectorSubcoreMesh(core_axis_name="core", subcore_axis_name="subcore")

    @pl.kernel(out_type=jax.ShapeDtypeStruct((r, d), cb.dtype), mesh=mesh)
    def gather_kernel(cb_hbm, i_hbm, o_hbm):
        def body(i_vmem, o_vmem):
            pltpu.sync_copy(cb_hbm.at[i_vmem.at[0]], o_vmem)

        pltpu.emit_pipeline(
            body,
            grid=(r // _GATHER_WINDOW,),
            in_specs=[pl.BlockSpec((1, _GATHER_WINDOW), index_map=lambda i: (0, i))],
            out_specs=[pl.BlockSpec((_GATHER_WINDOW, d), index_map=lambda i: (i, 0))],
            core_axis_name=("core", "subcore"),
            dimension_semantics=(pltpu.PARALLEL,),
        )(i_hbm, o_hbm)

    return gather_kernel(cb, idx2)


def _level_loss(md, numel):
    m = jnp.sum(md) / numel
    return _COMMIT * m + m


def kernel(coords, timestamps, attention_mask, W1, b1, W2, b2, cb0, cb1, cb2):
    feats = _build_point_features(coords, timestamps, attention_mask)
    h = jax.nn.gelu(feats @ W1 + b1, approximate=False)
    z = h @ W2 + b2
    b, t, d = z.shape

    z1 = _pool(z, _STRIDE1)
    z2 = _pool(z, _STRIDE2)
    f0 = z.reshape(-1, d)
    f1 = z1.reshape(-1, d)
    f2 = z2.reshape(-1, d)
    n0 = jnp.sum(f0 ** 2, axis=1, keepdims=True)
    n1 = jnp.sum(f1 ** 2, axis=1, keepdims=True)
    n2 = jnp.sum(f2 ** 2, axis=1, keepdims=True)
    c0n = jnp.sum(cb0 ** 2, axis=1)
    c1n = jnp.sum(cb1 ** 2, axis=1)
    c2n = jnp.sum(cb2 ** 2, axis=1)

    zneg0 = -2.0 * f0
    zneg2 = -2.0 * f2

    nsplit = 4
    qrows = f0.shape[0] // nsplit
    idx_parts, md_parts, g_parts = [], [], []
    for s in range(nsplit):
        idx_s, md_s = _vq_argmin(zneg0, n0, cb0, c0n, row_block=512,
                                 chunk=2048, row_start=s * qrows,
                                 row_count=qrows)
        g_parts.append(_sc_gather(cb0, idx_s))
        idx_parts.append(idx_s)
        md_parts.append(md_s)
    idx0f = jnp.concatenate(idx_parts)
    g0 = jnp.concatenate(g_parts)

    idx2f, md2 = _vq_argmin(zneg2, n2, cb2, c2n, row_block=512, chunk=512)

    dist1 = n1 - 2.0 * jax.lax.dot_general(
        f1, cb1, (((1,), (1,)), ((), ())),
        preferred_element_type=jnp.float32) + c1n
    idx1f = jnp.argmin(dist1, axis=1)
    q1 = jnp.take(cb1, idx1f, axis=0)
    l1 = _COMMIT * jnp.mean((q1 - f1) ** 2) + jnp.mean((q1 - f1) ** 2)

    q0 = (f0 + (g0 - f0)).reshape(z.shape)

    idx0 = idx0f.reshape(b, t)
    tok1 = _upsample(idx1f.reshape(b, -1), t)
    tok2 = _upsample(idx2f.reshape(b, -1), t)

    m0 = sum(jnp.sum(p) for p in md_parts) / f0.size
    l0 = _COMMIT * m0 + m0
    loss = l0 + l1 + _level_loss(md2, f2.size)
    return q0, idx0, tok1, tok2, loss
